# baseline (speedup 1.0000x reference)
import functools

import jax
import jax.numpy as jnp
from jax import lax
from jax.experimental import pallas as pl
from jax.experimental.pallas import tpu as pltpu
from jax.experimental.pallas import tpu_sc as plsc

N = 8192
D = 2048
HALF = N // 2
NUM_SUBCORES = 16
ROWS = HALF // NUM_SUBCORES
R = 16
C = ROWS // R
NSLOT = 2


def _body(x, out, buf, in_sems, out_sems):
    h = lax.axis_index("c")
    t = lax.axis_index("s")
    o0 = h * HALF + t * ROWS
    lane = lax.iota(jnp.int32, 16)

    def in_desc(k, slot):
        src_rows = (t * ROWS + k * R + lane) * 2 + h
        return pltpu.make_async_copy(x.at[src_rows], buf.at[slot], in_sems.at[slot])

    def out_desc(k, slot):
        return pltpu.make_async_copy(
            buf.at[slot], out.at[pl.ds(o0 + k * R, R)], out_sems.at[slot]
        )

    @pl.loop(0, C)
    def _chunk(g):
        slot = lax.rem(g, NSLOT)

        @pl.when(g >= NSLOT)
        def _():
            out_desc(g - NSLOT, slot).wait()

        in_desc(g, slot).start()

        @pl.when(g >= 1)
        def _():
            pslot = lax.rem(g - 1, NSLOT)
            in_desc(g - 1, pslot).wait()
            out_desc(g - 1, pslot).start()

    in_desc(C - 1, (C - 1) % NSLOT).wait()
    out_desc(C - 1, (C - 1) % NSLOT).start()
    for k in range(max(C - NSLOT + 1, 0), C):
        out_desc(k, k % NSLOT).wait()


@jax.jit
def _shuffle(x):
    mesh = plsc.VectorSubcoreMesh(core_axis_name="c", subcore_axis_name="s")
    return pl.kernel(
        _body,
        out_type=jax.ShapeDtypeStruct((N, D), jnp.float32),
        mesh=mesh,
        scratch_types=[
            pltpu.VMEM((NSLOT, R, D), jnp.float32),
            pltpu.SemaphoreType.DMA((NSLOT,)),
            pltpu.SemaphoreType.DMA((NSLOT,)),
        ],
    )(x)


def kernel(inputs):
    return _shuffle(inputs)


# speedup vs baseline: 1.0174x; 1.0174x over previous
import functools

import jax
import jax.numpy as jnp
from jax import lax
from jax.experimental import pallas as pl
from jax.experimental.pallas import tpu as pltpu
from jax.experimental.pallas import tpu_sc as plsc

N = 8192
D = 2048
HALF = N // 2
NUM_SUBCORES = 16
ROWS = HALF // NUM_SUBCORES
RB = 24
CB = 10
RT = ROWS - RB * CB
NSLOT = 2


def _body(x, out, buf, idx, in_sems, out_sems):
    h = lax.axis_index("c")
    t = lax.axis_index("s")
    o0 = h * HALF + t * ROWS
    lane = lax.iota(jnp.int32, 16)

    for k in range(CB + 1):
        base = (t * ROWS + RB * k) * 2 + h
        idx[k, pl.ds(0, 16)] = base + 2 * lane
        if k < CB:
            idx[k, pl.ds(8, 16)] = base + 2 * (lane + 8)

    def in_desc(k, slot, n):
        return pltpu.make_async_copy(
            x.at[idx.at[k, pl.ds(0, n)]],
            buf.at[slot, pl.ds(0, n)],
            in_sems.at[slot],
        )

    def out_desc(k, slot, n):
        return pltpu.make_async_copy(
            buf.at[slot, pl.ds(0, n)],
            out.at[pl.ds(o0 + RB * k, n)],
            out_sems.at[slot],
        )

    @pl.loop(0, CB)
    def _chunk(g):
        slot = lax.rem(g, NSLOT)

        @pl.when(g >= NSLOT)
        def _():
            out_desc(g - NSLOT, slot, RB).wait()

        in_desc(g, slot, RB).start()

        @pl.when(g >= 1)
        def _():
            in_desc(g - 1, 1 - slot, RB).wait()
            out_desc(g - 1, 1 - slot, RB).start()

    out_desc(CB - NSLOT, CB % NSLOT, RB).wait()
    in_desc(CB, CB % NSLOT, RT).start()
    in_desc(CB - 1, (CB - 1) % NSLOT, RB).wait()
    out_desc(CB - 1, (CB - 1) % NSLOT, RB).start()
    in_desc(CB, CB % NSLOT, RT).wait()
    out_desc(CB, CB % NSLOT, RT).start()
    out_desc(CB - 1, (CB - 1) % NSLOT, RB).wait()
    out_desc(CB, CB % NSLOT, RT).wait()


@jax.jit
def _shuffle(x):
    mesh = plsc.VectorSubcoreMesh(core_axis_name="c", subcore_axis_name="s")
    return pl.kernel(
        _body,
        out_type=jax.ShapeDtypeStruct((N, D), jnp.float32),
        mesh=mesh,
        scratch_types=[
            pltpu.VMEM((NSLOT, RB, D), jnp.float32),
            pltpu.VMEM((CB + 1, 32), jnp.int32),
            pltpu.SemaphoreType.DMA((NSLOT,)),
            pltpu.SemaphoreType.DMA((NSLOT,)),
        ],
    )(x)


def kernel(inputs):
    return _shuffle(inputs)
